# Initial kernel scaffold; baseline (speedup 1.0000x reference)
import jax
import jax.numpy as jnp
from jax.experimental import pallas as pl


def kernel(x, edge_index, pred_edges, W1, b1, W2, b2, Wl, bl):
    raise NotImplementedError("write your pallas kernel here")


# speedup vs baseline: 63.1989x; 63.1989x over previous
import functools

import jax
import jax.numpy as jnp
from jax import lax
from jax.experimental import pallas as pl
from jax.experimental.pallas import tpu as pltpu
from jax.experimental.pallas import tpu_sc as plsc

_N = 10000
_E = 320000
_P = 200000
_LANES = 16
_NW = 32
_EP = _E // _NW
_NV = _N // _LANES
_EV = _EP // _LANES
_PV = -(-(_P // _LANES) // _NW)
_PPT = _PV * _LANES
_P_PAD = _PPT * _NW

_mesh = plsc.VectorSubcoreMesh(core_axis_name="c", subcore_axis_name="s")
_sc_params = pltpu.CompilerParams(needs_layout_passes=False)


def _worker_id():
    return lax.axis_index("s") * 2 + lax.axis_index("c")


def _zero_vmem(ref, nv):
    def body(i, _):
        ref[pl.ds(i * _LANES, _LANES)] = jnp.zeros((_LANES,), jnp.float32)
        return _
    lax.fori_loop(0, nv, body, None)


@functools.partial(
    pl.kernel,
    out_type=jax.ShapeDtypeStruct((_NW, _N), jnp.float32),
    mesh=_mesh,
    compiler_params=_sc_params,
    scratch_types=[
        pltpu.VMEM((_EP,), jnp.int32),
        pltpu.VMEM((_N,), jnp.float32),
    ],
)
def _sc_degree(dst_hbm, out_hbm, dst_v, acc_v):
    w = _worker_id()
    pltpu.sync_copy(dst_hbm.at[pl.ds(w * _EP, _EP)], dst_v)
    _zero_vmem(acc_v, _NV)
    ones = jnp.ones((_LANES,), jnp.float32)

    def body(j, _):
        di = dst_v[pl.ds(j * _LANES, _LANES)]
        plsc.addupdate_scatter(acc_v, [di], ones)
        return _
    lax.fori_loop(0, _EV, body, None)
    pltpu.sync_copy(acc_v, out_hbm.at[w])


@functools.partial(
    pl.kernel,
    out_type=jax.ShapeDtypeStruct((_NW, _N), jnp.float32),
    mesh=_mesh,
    compiler_params=_sc_params,
    scratch_types=[
        pltpu.VMEM((_N,), jnp.float32),
        pltpu.VMEM((_EP,), jnp.int32),
        pltpu.VMEM((_EP,), jnp.int32),
        pltpu.VMEM((_N,), jnp.float32),
    ],
)
def _sc_scatter(vals_hbm, src_hbm, dst_hbm, out_hbm, vals_v, src_v, dst_v, acc_v):
    w = _worker_id()
    pltpu.sync_copy(vals_hbm, vals_v)
    pltpu.sync_copy(src_hbm.at[pl.ds(w * _EP, _EP)], src_v)
    pltpu.sync_copy(dst_hbm.at[pl.ds(w * _EP, _EP)], dst_v)
    _zero_vmem(acc_v, _NV)

    def body(j, _):
        b = j * _LANES
        si = src_v[pl.ds(b, _LANES)]
        di = dst_v[pl.ds(b, _LANES)]
        t = plsc.load_gather(vals_v, [si])
        plsc.addupdate_scatter(acc_v, [di], t)
        return _
    lax.fori_loop(0, _EV, body, None)
    pltpu.sync_copy(acc_v, out_hbm.at[w])


@functools.partial(
    pl.kernel,
    out_type=jax.ShapeDtypeStruct((_P_PAD,), jnp.float32),
    mesh=_mesh,
    compiler_params=_sc_params,
    scratch_types=[
        pltpu.VMEM((_N,), jnp.float32),
        pltpu.VMEM((2 * _PPT,), jnp.int32),
        pltpu.VMEM((_PPT,), jnp.float32),
    ],
)
def _sc_pair_gather(s_hbm, pe_hbm, out_hbm, s_v, pe_v, out_v):
    w = _worker_id()
    pltpu.sync_copy(s_hbm, s_v)
    pltpu.sync_copy(pe_hbm.at[pl.ds(w * 2 * _PPT, 2 * _PPT)], pe_v)
    iota2 = jnp.arange(_LANES, dtype=jnp.int32) * 2

    def body(j, _):
        ev = iota2 + j * (2 * _LANES)
        ia = plsc.load_gather(pe_v, [ev])
        ib = plsc.load_gather(pe_v, [ev + 1])
        va = plsc.load_gather(s_v, [ia])
        vb = plsc.load_gather(s_v, [ib])
        out_v[pl.ds(j * _LANES, _LANES)] = va * vb
        return _
    lax.fori_loop(0, _PV, body, None)
    pltpu.sync_copy(out_v, out_hbm.at[pl.ds(w * _PPT, _PPT)])


def _tc_mv_body(x_ref, w1_ref, w2_ref, wl_ref, u_ref):
    b = w1_ref[...] @ (w2_ref[...] @ wl_ref[...])
    u_ref[...] = jnp.dot(x_ref[...], b,
                         preferred_element_type=jnp.float32)


def _tc_prep_body(degp_ref, u_ref, ut_ref, dinv_ref):
    deg = jnp.sum(degp_ref[...], axis=0, keepdims=True) + 1.0
    dinv = lax.rsqrt(deg)
    dinv_ref[...] = dinv
    ut_ref[...] = u_ref[...] * dinv


def _tc_layer_body(p_ref, at_ref, dinv_ref, k_ref, out_ref, *, last):
    psum = jnp.sum(p_ref[...], axis=0, keepdims=True)
    dinv = dinv_ref[...]
    val = dinv * (psum + at_ref[...]) + k_ref[...]
    if last:
        out_ref[...] = 1.0 / (1.0 + jnp.exp(-val))
    else:
        out_ref[...] = val * dinv


def _tc_call(body, out_shapes, *args):
    return pl.pallas_call(
        body,
        out_shape=out_shapes,
    )(*args)


def kernel(x, edge_index, pred_edges, W1, b1, W2, b2, Wl, bl):
    f32 = jnp.float32
    src = edge_index[0]
    dst = edge_index[1]

    k1 = (b1 @ W2 @ Wl + 0.0).reshape(1, 1)
    k2 = (b2 @ Wl + bl).reshape(1, 1)

    degp = _sc_degree(dst)
    u = _tc_call(_tc_mv_body, jax.ShapeDtypeStruct((_N, 1), f32),
                 x, W1, W2, Wl)

    ut, dinv = _tc_call(
        _tc_prep_body,
        (jax.ShapeDtypeStruct((1, _N), f32), jax.ShapeDtypeStruct((1, _N), f32)),
        degp, u.reshape(1, _N))

    p1 = _sc_scatter(ut.reshape(_N), src, dst)
    gt = _tc_call(functools.partial(_tc_layer_body, last=False),
                  jax.ShapeDtypeStruct((1, _N), f32),
                  p1, ut, dinv, k1)

    p2 = _sc_scatter(gt.reshape(_N), src, dst)
    s = _tc_call(functools.partial(_tc_layer_body, last=True),
                 jax.ShapeDtypeStruct((1, _N), f32),
                 p2, gt, dinv, k2)

    pe_flat = jnp.pad(pred_edges.reshape(-1), (0, 2 * _P_PAD - 2 * _P))
    outp = _sc_pair_gather(s.reshape(_N), pe_flat)
    return outp[:_P]


# speedup vs baseline: 148.4969x; 2.3497x over previous
import functools

import jax
import jax.numpy as jnp
from jax import lax
from jax.experimental import pallas as pl
from jax.experimental.pallas import tpu as pltpu
from jax.experimental.pallas import tpu_sc as plsc

_N = 10000
_E = 320000
_P = 200000
_LANES = 16
_NW = 32
_EP = _E // _NW
_NV = _N // _LANES
_EV = _EP // _LANES
_EW = _EP + 112
_PV = -(-(_P // _LANES) // _NW)
_PPT = _PV * _LANES
_PPT_LAST = _P - (_NW - 1) * _PPT
_PV_LAST = _PPT_LAST // _LANES
_P_EXTRA = _PPT - _PPT_LAST
_PW = 6400
_PW_LAST = 6144
_P_PAD = 200064

_mesh = plsc.VectorSubcoreMesh(core_axis_name="c", subcore_axis_name="s")
_sc_params = pltpu.CompilerParams(needs_layout_passes=False)


def _worker_id():
    return lax.axis_index("s") * 2 + lax.axis_index("c")


def _zero_vmem(ref, nv):
    def body(i, _):
        ref[pl.ds(i * _LANES, _LANES)] = jnp.zeros((_LANES,), jnp.float32)
        return _
    lax.fori_loop(0, nv, body, None)


@functools.partial(
    pl.kernel,
    out_type=jax.ShapeDtypeStruct((_NW, _N), jnp.float32),
    mesh=_mesh,
    compiler_params=_sc_params,
    scratch_types=[
        pltpu.VMEM((2, _EW), jnp.int32),
        pltpu.VMEM((_N,), jnp.float32),
    ],
)
def _sc_degree(ei_hbm, out_hbm, ei_v, acc_v):
    w = _worker_id()
    start = w * _EP
    lead = lax.rem(start, 128)
    ab = pl.multiple_of(start - lead, 128)
    pltpu.sync_copy(ei_hbm.at[:, pl.ds(ab, _EW)], ei_v)
    _zero_vmem(acc_v, _NV)
    ones = jnp.ones((_LANES,), jnp.float32)

    def body(j, _):
        di = ei_v[1, pl.ds(lead + j * _LANES, _LANES)]
        plsc.addupdate_scatter(acc_v, [di], ones)
        return _
    lax.fori_loop(0, _EV, body, None)
    pltpu.sync_copy(acc_v, out_hbm.at[w])


@functools.partial(
    pl.kernel,
    out_type=jax.ShapeDtypeStruct((_NW, _N), jnp.float32),
    mesh=_mesh,
    compiler_params=_sc_params,
    scratch_types=[
        pltpu.VMEM((_N,), jnp.float32),
        pltpu.VMEM((2, _EW), jnp.int32),
        pltpu.VMEM((_N,), jnp.float32),
    ],
)
def _sc_scatter(vals_hbm, ei_hbm, out_hbm, vals_v, ei_v, acc_v):
    w = _worker_id()
    start = w * _EP
    lead = lax.rem(start, 128)
    ab = pl.multiple_of(start - lead, 128)
    pltpu.sync_copy(vals_hbm, vals_v)
    pltpu.sync_copy(ei_hbm.at[:, pl.ds(ab, _EW)], ei_v)
    _zero_vmem(acc_v, _NV)

    def body(j, _):
        b = lead + j * _LANES
        si = ei_v[0, pl.ds(b, _LANES)]
        di = ei_v[1, pl.ds(b, _LANES)]
        t = plsc.load_gather(vals_v, [si])
        plsc.addupdate_scatter(acc_v, [di], t)
        return _
    lax.fori_loop(0, _EV, body, None)
    pltpu.sync_copy(acc_v, out_hbm.at[w])


@functools.partial(
    pl.kernel,
    out_type=jax.ShapeDtypeStruct((_P,), jnp.float32),
    mesh=_mesh,
    compiler_params=_sc_params,
    scratch_types=[
        pltpu.VMEM((_N,), jnp.float32),
        pltpu.VMEM((2, _PW), jnp.int32),
        pltpu.VMEM((_PPT,), jnp.float32),
    ],
)
def _sc_pair_gather(s_hbm, pe_hbm, out_hbm, s_v, pe_v, out_v):
    w = _worker_id()
    start = w * _PPT
    lead = lax.rem(start, 128)
    ab = pl.multiple_of(start - lead, 128)
    pltpu.sync_copy(s_hbm, s_v)

    @pl.when(w < _NW - 1)
    def _():
        pltpu.sync_copy(pe_hbm.at[:, pl.ds(ab, _PW)], pe_v.at[:, pl.ds(0, _PW)])

    @pl.when(w == _NW - 1)
    def _():
        pltpu.sync_copy(pe_hbm.at[:, pl.ds(ab, _PW_LAST)],
                        pe_v.at[:, pl.ds(0, _PW_LAST)])

    def body(j, _):
        b = lead + j * _LANES
        ia = pe_v[0, pl.ds(b, _LANES)]
        ib = pe_v[1, pl.ds(b, _LANES)]
        va = plsc.load_gather(s_v, [ia])
        vb = plsc.load_gather(s_v, [ib])
        out_v[pl.ds(j * _LANES, _LANES)] = va * vb
        return _
    lax.fori_loop(0, _PV_LAST, body, None)

    @pl.when(w < _NW - 1)
    def _():
        lax.fori_loop(_PV_LAST, _PV, body, None)

    pltpu.sync_copy(out_v.at[pl.ds(0, _PPT_LAST)],
                    out_hbm.at[pl.ds(start, _PPT_LAST)])

    @pl.when(w < _NW - 1)
    def _():
        pltpu.sync_copy(out_v.at[pl.ds(_PPT_LAST, _P_EXTRA)],
                        out_hbm.at[pl.ds(start + _PPT_LAST, _P_EXTRA)])


def _tc_mv_body(x_ref, w1_ref, w2_ref, wl_ref, u_ref):
    b = w1_ref[...] @ (w2_ref[...] @ wl_ref[...])
    u_ref[...] = jnp.dot(x_ref[...], b,
                         preferred_element_type=jnp.float32)


def _tc_prep_body(degp_ref, u_ref, ut_ref, dinv_ref):
    deg = jnp.sum(degp_ref[...], axis=0, keepdims=True) + 1.0
    dinv = lax.rsqrt(deg)
    dinv_ref[...] = dinv
    ut_ref[...] = u_ref[...] * dinv


def _tc_layer_body(p_ref, at_ref, dinv_ref, k_ref, out_ref, *, last):
    psum = jnp.sum(p_ref[...], axis=0, keepdims=True)
    dinv = dinv_ref[...]
    val = dinv * (psum + at_ref[...]) + k_ref[...]
    if last:
        out_ref[...] = 1.0 / (1.0 + jnp.exp(-val))
    else:
        out_ref[...] = val * dinv


def _tc_call(body, out_shapes, *args):
    return pl.pallas_call(
        body,
        out_shape=out_shapes,
    )(*args)


def kernel(x, edge_index, pred_edges, W1, b1, W2, b2, Wl, bl):
    f32 = jnp.float32

    k1 = (b1 @ W2 @ Wl + 0.0).reshape(1, 1)
    k2 = (b2 @ Wl + bl).reshape(1, 1)

    degp = _sc_degree(edge_index)
    u = _tc_call(_tc_mv_body, jax.ShapeDtypeStruct((_N, 1), f32),
                 x, W1, W2, Wl)

    ut, dinv = _tc_call(
        _tc_prep_body,
        (jax.ShapeDtypeStruct((1, _N), f32), jax.ShapeDtypeStruct((1, _N), f32)),
        degp, u.reshape(1, _N))

    p1 = _sc_scatter(ut.reshape(_N), edge_index)
    gt = _tc_call(functools.partial(_tc_layer_body, last=False),
                  jax.ShapeDtypeStruct((1, _N), f32),
                  p1, ut, dinv, k1)

    p2 = _sc_scatter(gt.reshape(_N), edge_index)
    s = _tc_call(functools.partial(_tc_layer_body, last=True),
                 jax.ShapeDtypeStruct((1, _N), f32),
                 p2, gt, dinv, k2)

    pe_t = jnp.pad(pred_edges.T, ((0, 0), (0, _P_PAD - _P)))
    return _sc_pair_gather(s.reshape(_N), pe_t)
